# initial kernel scaffold (unmeasured)
import jax
import jax.numpy as jnp
from jax import lax
from jax.experimental import pallas as pl
from jax.experimental.pallas import tpu as pltpu

N_DEV = 4
SQ = 2048
SKV = 2048
D_MODEL = 1024
HPD = 8
DH = 128
BLK = HPD * DH
SCALE = 0.08838834764831843
Q_CHUNK = 1024
NEG = -1e9


def kernel(x, Wq, K_ext, V_ext, Wo):
    def body(x_ref, wq_ref, k_hbm, v_hbm, wo_ref, out_ref,
             comm, k_s, v_s, q_s, ctx_s, send_sems, recv_sems, copy_sems):
        my_pos = lax.axis_index("i")
        right = lax.rem(my_pos + 1, N_DEV)
        left = lax.rem(my_pos + N_DEV - 1, N_DEV)

        barrier_sem = pltpu.get_barrier_semaphore()
        for nbr in (left, right):
            pl.semaphore_signal(barrier_sem, inc=1, device_id=(nbr,),
                                device_id_type=pl.DeviceIdType.MESH)
        pl.semaphore_wait(barrier_sem, 2)

        def hop_rdmas(h, src_q, src_o):
            rq = pltpu.make_async_remote_copy(
                src_ref=src_q, dst_ref=comm.at[h, 0],
                send_sem=send_sems.at[h, 0], recv_sem=recv_sems.at[h, 0],
                device_id=(right,), device_id_type=pl.DeviceIdType.MESH)
            ro = pltpu.make_async_remote_copy(
                src_ref=src_o, dst_ref=comm.at[h, 1],
                send_sem=send_sems.at[h, 1], recv_sem=recv_sems.at[h, 1],
                device_id=(right,), device_id_type=pl.DeviceIdType.MESH)
            return rq, ro

        def compute_block(g, wq_blk, wo_blk, first):
            ck = pltpu.make_async_copy(
                k_hbm.at[my_pos, :, pl.ds(g * HPD, HPD), :], k_s,
                copy_sems.at[0])
            cv = pltpu.make_async_copy(
                v_hbm.at[my_pos, :, pl.ds(g * HPD, HPD), :], v_s,
                copy_sems.at[1])
            ck.start()
            cv.start()
            ck.wait()
            cv.wait()

            q_s[...] = jnp.dot(x_ref[0], wq_blk,
                               preferred_element_type=jnp.float32)
            for h in range(HPD):
                k_h = k_s[:, h, :]
                v_h = v_s[:, h, :]
                q_h = q_s[:, h * DH:(h + 1) * DH]
                for qc in range(SQ // Q_CHUNK):
                    qs = qc * Q_CHUNK
                    scores = lax.dot_general(
                        q_h[qs:qs + Q_CHUNK], k_h,
                        (((1,), (1,)), ((), ())),
                        preferred_element_type=jnp.float32) * SCALE
                    qb = (lax.broadcasted_iota(jnp.int32, (Q_CHUNK, SKV), 0)
                          + qs) // 64
                    kb = lax.broadcasted_iota(jnp.int32, (Q_CHUNK, SKV),
                                              1) // 64
                    mask = (qb == kb) | (kb == 0) | (lax.rem(qb + kb, 3) == 0)
                    scores = jnp.where(mask, scores, NEG)
                    m = jnp.max(scores, axis=1, keepdims=True)
                    w = jnp.exp(scores - m)
                    w = w / jnp.sum(w, axis=1, keepdims=True)
                    ctx_s[qs:qs + Q_CHUNK, h * DH:(h + 1) * DH] = jnp.dot(
                        w, v_h, preferred_element_type=jnp.float32)
            acc = jnp.dot(ctx_s[...], wo_blk,
                          preferred_element_type=jnp.float32)
            if first:
                out_ref[0, :, :] = acc
            else:
                out_ref[0, :, :] = out_ref[0, :, :] + acc

        rq, ro = hop_rdmas(0, wq_ref, wo_ref)
        rq.start()
        ro.start()
        compute_block(my_pos, wq_ref[...], wo_ref[...], first=True)

        for h in range(N_DEV - 1):
            rq.wait()
            ro.wait()
            if h < N_DEV - 2:
                rq, ro = hop_rdmas(h + 1, comm.at[h, 0], comm.at[h, 1])
                rq.start()
                ro.start()
            g = lax.rem(my_pos + (N_DEV - 1 - h), N_DEV)
            compute_block(g, comm[h, 0], comm[h, 1], first=False)

    return pl.pallas_call(
        body,
        out_shape=jax.ShapeDtypeStruct((1, SQ, D_MODEL), jnp.float32),
        in_specs=[
            pl.BlockSpec(memory_space=pltpu.VMEM),
            pl.BlockSpec(memory_space=pltpu.VMEM),
            pl.BlockSpec(memory_space=pltpu.ANY),
            pl.BlockSpec(memory_space=pltpu.ANY),
            pl.BlockSpec(memory_space=pltpu.VMEM),
        ],
        out_specs=pl.BlockSpec(memory_space=pltpu.VMEM),
        scratch_shapes=[
            pltpu.VMEM((N_DEV - 1, 2, D_MODEL, BLK), jnp.float32),
            pltpu.VMEM((SKV, HPD, DH), jnp.float32),
            pltpu.VMEM((SKV, HPD, DH), jnp.float32),
            pltpu.VMEM((SQ, BLK), jnp.float32),
            pltpu.VMEM((SQ, BLK), jnp.float32),
            pltpu.SemaphoreType.DMA((N_DEV - 1, 2)),
            pltpu.SemaphoreType.DMA((N_DEV - 1, 2)),
            pltpu.SemaphoreType.DMA((2,)),
        ],
        compiler_params=pltpu.CompilerParams(collective_id=0),
    )(x, Wq, K_ext, V_ext, Wo)


# baseline (device time: 1418666 ns/iter reference)
import jax
import jax.numpy as jnp
from jax import lax
from jax.experimental import pallas as pl
from jax.experimental.pallas import tpu as pltpu

N_DEV = 4
SQ = 2048
SKV = 2048
D_MODEL = 1024
HPD = 8
HHALF = 2
DH = 128
BLK = HPD * DH
SCALE = 0.08838834764831843
QC = 128
NEG = -1e9


def kernel(x, Wq, K_ext, V_ext, Wo):
    def body(x_ref, wq_ref, k_hbm, v_hbm, wo_ref, out_ref,
             comm, k_s, v_s, ctx_s, send_sems, recv_sems, copy_sems,
             credit_sem):
        my_pos = lax.axis_index("i")
        right = lax.rem(my_pos + 1, N_DEV)
        left = lax.rem(my_pos + N_DEV - 1, N_DEV)

        barrier_sem = pltpu.get_barrier_semaphore()
        for nbr in (left, right):
            pl.semaphore_signal(barrier_sem, inc=1, device_id=(nbr,),
                                device_id_type=pl.DeviceIdType.MESH)
        pl.semaphore_wait(barrier_sem, 2)

        def hop_rdmas(h, src_q, src_o):
            slot = h % 2
            rq = pltpu.make_async_remote_copy(
                src_ref=src_q, dst_ref=comm.at[slot, 0],
                send_sem=send_sems.at[h, 0], recv_sem=recv_sems.at[h, 0],
                device_id=(right,), device_id_type=pl.DeviceIdType.MESH)
            ro = pltpu.make_async_remote_copy(
                src_ref=src_o, dst_ref=comm.at[slot, 1],
                send_sem=send_sems.at[h, 1], recv_sem=recv_sems.at[h, 1],
                device_id=(right,), device_id_type=pl.DeviceIdType.MESH)
            return rq, ro

        def compute_block(g, wq_v, wo_v, first):
            for half in range(HPD // HHALF):
                h0 = half * HHALF
                ck = pltpu.make_async_copy(
                    k_hbm.at[my_pos, :, pl.ds(g * HPD + h0, HHALF), :],
                    k_s, copy_sems.at[0])
                cv = pltpu.make_async_copy(
                    v_hbm.at[my_pos, :, pl.ds(g * HPD + h0, HHALF), :],
                    v_s, copy_sems.at[1])
                ck.start()
                cv.start()
                ck.wait()
                cv.wait()

                def qc_body(qc, carry):
                    qs = qc * QC
                    x_chunk = x_ref[0, pl.ds(qs, QC), :]
                    for hh in range(HHALF):
                        h = h0 + hh
                        q_c = jnp.dot(
                            x_chunk, wq_v[:, h * DH:(h + 1) * DH],
                            preferred_element_type=jnp.float32)
                        scores = lax.dot_general(
                            q_c, k_s[:, hh, :],
                            (((1,), (1,)), ((), ())),
                            preferred_element_type=jnp.float32) * SCALE
                        qb = (lax.broadcasted_iota(jnp.int32, (QC, SKV), 0)
                              + qs) // 64
                        kb = lax.broadcasted_iota(jnp.int32, (QC, SKV),
                                                  1) // 64
                        mask = ((qb == kb) | (kb == 0)
                                | (lax.rem(qb + kb, 3) == 0))
                        scores = jnp.where(mask, scores, NEG)
                        m = jnp.max(scores, axis=1, keepdims=True)
                        w = jnp.exp(scores - m)
                        w = w / jnp.sum(w, axis=1, keepdims=True)
                        ctx_s[:, hh * DH:(hh + 1) * DH] = jnp.dot(
                            w, v_s[:, hh, :],
                            preferred_element_type=jnp.float32)
                    o = jnp.dot(
                        ctx_s[...],
                        wo_v[h0 * DH:(h0 + HHALF) * DH, :],
                        preferred_element_type=jnp.float32)
                    if first and half == 0:
                        out_ref[0, pl.ds(qs, QC), :] = o
                    else:
                        out_ref[0, pl.ds(qs, QC), :] = (
                            out_ref[0, pl.ds(qs, QC), :] + o)
                    return carry

                lax.fori_loop(0, SQ // QC, qc_body, None)

        rq, ro = hop_rdmas(0, wq_ref, wo_ref)
        rq.start()
        ro.start()
        compute_block(my_pos, wq_ref, wo_ref, first=True)

        for h in range(N_DEV - 1):
            rq.wait()
            ro.wait()
            if h < N_DEV - 2:
                if h + 1 == 2:
                    pl.semaphore_signal(
                        credit_sem, inc=1, device_id=(left,),
                        device_id_type=pl.DeviceIdType.MESH)
                    pl.semaphore_wait(credit_sem, 1)
                slot = h % 2
                rq, ro = hop_rdmas(h + 1, comm.at[slot, 0],
                                   comm.at[slot, 1])
                rq.start()
                ro.start()
            slot = h % 2
            g = lax.rem(my_pos + (N_DEV - 1 - h), N_DEV)
            compute_block(g, comm.at[slot, 0], comm.at[slot, 1],
                          first=False)

    return pl.pallas_call(
        body,
        out_shape=jax.ShapeDtypeStruct((1, SQ, D_MODEL), jnp.float32),
        in_specs=[
            pl.BlockSpec(memory_space=pltpu.VMEM),
            pl.BlockSpec(memory_space=pltpu.VMEM),
            pl.BlockSpec(memory_space=pl.ANY),
            pl.BlockSpec(memory_space=pl.ANY),
            pl.BlockSpec(memory_space=pltpu.VMEM),
        ],
        out_specs=pl.BlockSpec(memory_space=pltpu.VMEM),
        scratch_shapes=[
            pltpu.VMEM((2, 2, D_MODEL, BLK), jnp.float32),
            pltpu.VMEM((SKV, HHALF, DH), jnp.float32),
            pltpu.VMEM((SKV, HHALF, DH), jnp.float32),
            pltpu.VMEM((QC, HHALF * DH), jnp.float32),
            pltpu.SemaphoreType.DMA((N_DEV - 1, 2)),
            pltpu.SemaphoreType.DMA((N_DEV - 1, 2)),
            pltpu.SemaphoreType.DMA((2,)),
            pltpu.SemaphoreType.REGULAR,
        ],
        compiler_params=pltpu.CompilerParams(
            collective_id=0, vmem_limit_bytes=100 * 1024 * 1024),
    )(x, Wq, K_ext, V_ext, Wo)
